# baseline (speedup 1.0000x reference)
import functools

import jax
import jax.numpy as jnp
from jax import lax
from jax.experimental import pallas as pl
from jax.experimental.pallas import tpu as pltpu
from jax.experimental.pallas import tpu_sc as plsc

_NW = 32


def _gather_body(rows_per_w, n_pairs, ids_hbm, table_hbm, out_hbm,
                 idx2, rows0, rows1, gs0, gs1, os0, os1):
    wid = lax.axis_index("s") * 2 + lax.axis_index("c")
    r0 = wid * rows_per_w
    pltpu.sync_copy(ids_hbm.at[pl.ds(r0, rows_per_w)], idx2)

    def g_copy(g, rows, sem):
        return pltpu.make_async_copy(table_hbm.at[idx2.at[g]], rows, sem)

    def s_copy(g, rows, sem):
        return pltpu.make_async_copy(rows, out_hbm.at[r0 + g], sem)

    g_copy(0, rows0, gs0).start()

    def body(i, carry):
        a = 2 * i
        g_copy(a, rows0, gs0).wait()
        s_copy(a, rows0, os0).start()

        @pl.when(i > 0)
        def _():
            s_copy(a - 1, rows1, os1).wait()

        g_copy(a + 1, rows1, gs1).start()
        g_copy(a + 1, rows1, gs1).wait()
        s_copy(a + 1, rows1, os1).start()

        @pl.when(i + 1 < n_pairs)
        def _():
            s_copy(a, rows0, os0).wait()
            g_copy(a + 2, rows0, gs0).start()

        return carry

    lax.fori_loop(0, n_pairs, body, 0, unroll=False)
    s_copy(0, rows0, os0).wait()
    s_copy(0, rows1, os1).wait()


def kernel(token_ids, table):
    b, s = token_ids.shape
    _, d = table.shape
    assert b % (_NW * 2) == 0
    rows_per_w = b // _NW
    n_pairs = rows_per_w // 2

    mesh = plsc.VectorSubcoreMesh(core_axis_name="c", subcore_axis_name="s")
    k = pl.kernel(
        functools.partial(_gather_body, rows_per_w, n_pairs),
        out_type=jax.ShapeDtypeStruct((b, s, d), jnp.float32),
        mesh=mesh,
        scratch_types=[
            pltpu.VMEM((rows_per_w, s), jnp.int32),
            pltpu.VMEM((s, d), jnp.float32),
            pltpu.VMEM((s, d), jnp.float32),
            pltpu.SemaphoreType.DMA,
            pltpu.SemaphoreType.DMA,
            pltpu.SemaphoreType.DMA,
            pltpu.SemaphoreType.DMA,
        ],
        compiler_params=pltpu.CompilerParams(use_tc_tiling_on_sc=False),
    )
    return k(token_ids, table)
